# baseline (speedup 1.0000x reference)
import dataclasses
import functools

import jax
import jax.numpy as jnp
from jax import lax
from jax.experimental import pallas as pl
from jax.experimental.pallas import tpu as pltpu
from jax.experimental.pallas import tpu_sc as plsc

NUM_CLASSES = 80
MAX_BOXES = 100
N = 60
L = 16
NVEC = 4


def _f32(*shape):
    return jax.ShapeDtypeStruct(shape, jnp.float32)


_MESH = plsc.VectorSubcoreMesh(core_axis_name="c", subcore_axis_name="s")

_CP = pltpu.CompilerParams()
if "needs_layout_passes" in pltpu.CompilerParams.__dataclass_fields__:
    _CP = dataclasses.replace(_CP, needs_layout_passes=False)
if "use_tc_tiling_on_sc" in pltpu.CompilerParams.__dataclass_fields__:
    _CP = dataclasses.replace(_CP, use_tc_tiling_on_sc=True)
_CP = dataclasses.replace(_CP, internal_scratch_in_bytes=1 << 20)


@functools.partial(
    pl.kernel,
    out_type=(
        _f32(4, 64),
        _f32(64, 64), _f32(64, 64, NUM_CLASSES),
        _f32(32, 32), _f32(32, 32, NUM_CLASSES),
        _f32(16, 16), _f32(16, 16, NUM_CLASSES),
    ),
    mesh=_MESH,
    compiler_params=_CP,
    scratch_types=[
        pltpu.VMEM((N, 5), jnp.float32),
        pltpu.VMEM((8, NVEC * L), jnp.int32),
        pltpu.VMEM((2, 64, NUM_CLASSES), jnp.float32),
        pltpu.VMEM((1, 32, NUM_CLASSES), jnp.float32),
        pltpu.VMEM((1, 16, NUM_CLASSES), jnp.float32),
        pltpu.VMEM((2, 64), jnp.float32),
        pltpu.VMEM((1, 32), jnp.float32),
        pltpu.VMEM((1, 16), jnp.float32),
        pltpu.VMEM((4, NVEC * L), jnp.float32),
        pltpu.SemaphoreType.DMA,
    ],
)
def _sc_label_kernel(label_hbm,
                     stage_hbm, obj8_hbm, cls8_hbm,
                     obj16_hbm, cls16_hbm, obj32_hbm, cls32_hbm,
                     lab_v, ci_v, c8_v, c16_v, c32_v, o8_v, o16_v, o32_v,
                     stage_v, sem):
    w = lax.axis_index("s") * 2 + lax.axis_index("c")
    x0 = 2 * w

    iota = lax.iota(jnp.int32, L)
    zeros16 = jnp.zeros((L,), jnp.float32)
    ones16 = jnp.ones((L,), jnp.float32)
    z16i = jnp.zeros((L,), jnp.int32)

    pltpu.sync_copy(label_hbm, lab_v)

    @pl.loop(0, NVEC)
    def _(v):
        rows = jnp.minimum(iota + v * L, N - 1)

        def _col(c):
            return plsc.load_gather(lab_v, [rows, jnp.full((L,), c, jnp.int32)])

        c8x = (_col(1) * 0.125).astype(jnp.int32)
        c8y = (_col(2) * 0.125).astype(jnp.int32)
        c16x = lax.shift_right_logical(c8x, 4)
        c16y = lax.shift_right_logical(c8y, 4)
        ci_v[0, pl.ds(v * L, L)] = c8x
        ci_v[1, pl.ds(v * L, L)] = c8y
        ci_v[2, pl.ds(v * L, L)] = c16x
        ci_v[3, pl.ds(v * L, L)] = c16y
        ci_v[4, pl.ds(v * L, L)] = lax.shift_right_logical(c16x, 5)
        ci_v[5, pl.ds(v * L, L)] = lax.shift_right_logical(c16y, 5)
        ci_v[6, pl.ds(v * L, L)] = _col(0).astype(jnp.int32)

    def _scatter_pass(cls_ref, obj_ref, rx, ry, xlo, nrows):
        @pl.loop(0, NVEC)
        def _(v):
            cx = ci_v[rx, pl.ds(v * L, L)]
            inr = (iota < (N - v * L)) & (cx >= xlo) & (cx < xlo + nrows)

            @pl.when(jnp.any(inr))
            def _():
                cy = ci_v[ry, pl.ds(v * L, L)]
                cls_i = ci_v[6, pl.ds(v * L, L)]
                lx = jnp.minimum(jnp.maximum(cx - xlo, 0), nrows - 1)

                @pl.loop(0, L)
                def _(l):
                    m = (iota == l) & inr
                    plsc.addupdate_scatter(cls_ref, [lx, cy, cls_i],
                                           ones16, mask=m)
                    plsc.addupdate_scatter(obj_ref, [lx, cy], ones16, mask=m)

    handles = []

    @pl.loop(0, 2)
    def _(i):
        @pl.loop(0, 64)
        def _(j):
            for k in range(NUM_CLASSES // L):
                c8_v[i, j, pl.ds(k * L, L)] = zeros16

    for i in range(2):
        for k in range(64 // L):
            o8_v[i, pl.ds(k * L, L)] = zeros16

    _scatter_pass(c8_v, o8_v, 0, 1, x0, 2)
    handles.append(pltpu.async_copy(c8_v, cls8_hbm.at[pl.ds(x0, 2)], sem))
    handles.append(pltpu.async_copy(o8_v, obj8_hbm.at[pl.ds(x0, 2)], sem))

    @pl.loop(0, 32)
    def _(j):
        for k in range(NUM_CLASSES // L):
            c16_v[0, j, pl.ds(k * L, L)] = zeros16

    for k in range(32 // L):
        o16_v[0, pl.ds(k * L, L)] = zeros16

    _scatter_pass(c16_v, o16_v, 2, 3, w, 1)
    handles.append(pltpu.async_copy(c16_v, cls16_hbm.at[pl.ds(w, 1)], sem))
    handles.append(pltpu.async_copy(o16_v, obj16_hbm.at[pl.ds(w, 1)], sem))

    @pl.when(w < 16)
    def _():
        @pl.loop(0, 16)
        def _(j):
            for k in range(NUM_CLASSES // L):
                c32_v[0, j, pl.ds(k * L, L)] = zeros16

        o32_v[0, pl.ds(0, L)] = zeros16

        _scatter_pass(c32_v, o32_v, 4, 5, w, 1)
        pltpu.sync_copy(c32_v, cls32_hbm.at[pl.ds(w, 1)])
        pltpu.sync_copy(o32_v, obj32_hbm.at[pl.ds(w, 1)])

    @pl.when(w == 17)
    def _():
        @pl.loop(0, NVEC)
        def _(v):
            for r in range(4):
                stage_v[r, pl.ds(v * L, L)] = (
                    ci_v[r, pl.ds(v * L, L)].astype(jnp.float32))
        pltpu.sync_copy(stage_v, stage_hbm)

    for h in handles:
        h.wait()


def _tc_tail_kernel(image_ref, label_ref, stage_ref,
                    img_ref, box8_ref, box16_ref, box32_ref):
    img_ref[...] = image_ref[...]

    @pl.when(pl.program_id(0) == 0)
    def _():
        lab = label_ref[...]
        stage = stage_ref[...]
        pad = jnp.zeros((MAX_BOXES - N, 4), dtype=jnp.float32)

        def box(c0, c1):
            cols = jnp.concatenate(
                [c0.reshape(N, 1), c1.reshape(N, 1), lab[:, 3:5]], axis=1)
            return jnp.concatenate([cols, pad], axis=0)

        box8_ref[...] = box(lab[:, 1], lab[:, 2])
        box16_ref[...] = box(stage[0, :N], stage[1, :N])
        box32_ref[...] = box(stage[2, :N], stage[3, :N])


def kernel(image, label):
    (stage, obj8, cls8, obj16, cls16,
     obj32, cls32) = _sc_label_kernel(label)
    img, box8, box16, box32 = pl.pallas_call(
        _tc_tail_kernel,
        grid=(2,),
        in_specs=[
            pl.BlockSpec((3, 256, 512), lambda i: (0, i, 0)),
            pl.BlockSpec((N, 5), lambda i: (0, 0)),
            pl.BlockSpec((4, 64), lambda i: (0, 0)),
        ],
        out_specs=[
            pl.BlockSpec((3, 256, 512), lambda i: (0, i, 0)),
            pl.BlockSpec((MAX_BOXES, 4), lambda i: (0, 0)),
            pl.BlockSpec((MAX_BOXES, 4), lambda i: (0, 0)),
            pl.BlockSpec((MAX_BOXES, 4), lambda i: (0, 0)),
        ],
        out_shape=(
            jax.ShapeDtypeStruct((3, 512, 512), jnp.float32),
            _f32(MAX_BOXES, 4), _f32(MAX_BOXES, 4), _f32(MAX_BOXES, 4),
        ),
        compiler_params=pltpu.CompilerParams(
            dimension_semantics=("parallel",)),
    )(image, label, stage)
    return (img, box8, obj8, cls8, box16, obj16, cls16,
            box32, obj32, cls32)


# speedup vs baseline: 1.0298x; 1.0298x over previous
import dataclasses
import functools

import jax
import jax.numpy as jnp
from jax import lax
from jax.experimental import pallas as pl
from jax.experimental.pallas import tpu as pltpu
from jax.experimental.pallas import tpu_sc as plsc

NUM_CLASSES = 80
MAX_BOXES = 100
N = 60
L = 16
NVEC = 4


def _f32(*shape):
    return jax.ShapeDtypeStruct(shape, jnp.float32)


_MESH = plsc.VectorSubcoreMesh(core_axis_name="c", subcore_axis_name="s")

_CP = pltpu.CompilerParams()
if "needs_layout_passes" in pltpu.CompilerParams.__dataclass_fields__:
    _CP = dataclasses.replace(_CP, needs_layout_passes=False)
if "use_tc_tiling_on_sc" in pltpu.CompilerParams.__dataclass_fields__:
    _CP = dataclasses.replace(_CP, use_tc_tiling_on_sc=True)
_CP = dataclasses.replace(_CP, internal_scratch_in_bytes=1 << 20)


@functools.partial(
    pl.kernel,
    out_type=(
        _f32(4, 64),
        _f32(64, 64), _f32(64, 64, NUM_CLASSES),
        _f32(32, 32), _f32(32, 32, NUM_CLASSES),
        _f32(16, 16), _f32(16, 16, NUM_CLASSES),
    ),
    mesh=_MESH,
    compiler_params=_CP,
    scratch_types=[
        pltpu.VMEM((N, 5), jnp.float32),
        pltpu.VMEM((8, NVEC * L), jnp.int32),
        pltpu.VMEM((2, 64, NUM_CLASSES), jnp.float32),
        pltpu.VMEM((1, 32, NUM_CLASSES), jnp.float32),
        pltpu.VMEM((1, 16, NUM_CLASSES), jnp.float32),
        pltpu.VMEM((2, 64), jnp.float32),
        pltpu.VMEM((1, 32), jnp.float32),
        pltpu.VMEM((1, 16), jnp.float32),
        pltpu.VMEM((4, NVEC * L), jnp.float32),
        pltpu.SemaphoreType.DMA,
    ],
)
def _sc_label_kernel(label_hbm,
                     stage_hbm, obj8_hbm, cls8_hbm,
                     obj16_hbm, cls16_hbm, obj32_hbm, cls32_hbm,
                     lab_v, ci_v, c8_v, c16_v, c32_v, o8_v, o16_v, o32_v,
                     stage_v, sem):
    w = lax.axis_index("s") * 2 + lax.axis_index("c")
    x0 = 2 * w

    iota = lax.iota(jnp.int32, L)
    zeros16 = jnp.zeros((L,), jnp.float32)
    ones16 = jnp.ones((L,), jnp.float32)
    z16i = jnp.zeros((L,), jnp.int32)

    pltpu.sync_copy(label_hbm, lab_v)

    @pl.loop(0, NVEC)
    def _(v):
        rows = jnp.minimum(iota + v * L, N - 1)

        def _col(c):
            return plsc.load_gather(lab_v, [rows, jnp.full((L,), c, jnp.int32)])

        c8x = (_col(1) * 0.125).astype(jnp.int32)
        c8y = (_col(2) * 0.125).astype(jnp.int32)
        c16x = lax.shift_right_logical(c8x, 4)
        c16y = lax.shift_right_logical(c8y, 4)
        ci_v[0, pl.ds(v * L, L)] = c8x
        ci_v[1, pl.ds(v * L, L)] = c8y
        ci_v[2, pl.ds(v * L, L)] = c16x
        ci_v[3, pl.ds(v * L, L)] = c16y
        ci_v[4, pl.ds(v * L, L)] = lax.shift_right_logical(c16x, 5)
        ci_v[5, pl.ds(v * L, L)] = lax.shift_right_logical(c16y, 5)
        ci_v[6, pl.ds(v * L, L)] = _col(0).astype(jnp.int32)

    def _scatter_pass(cls_ref, obj_ref, rx, ry, xlo, nrows):
        @pl.loop(0, NVEC)
        def _(v):
            cx = ci_v[rx, pl.ds(v * L, L)]
            inr = (iota < (N - v * L)) & (cx >= xlo) & (cx < xlo + nrows)

            @pl.when(jnp.any(inr))
            def _():
                cy = ci_v[ry, pl.ds(v * L, L)]
                cls_i = ci_v[6, pl.ds(v * L, L)]
                lx = jnp.minimum(jnp.maximum(cx - xlo, 0), nrows - 1)

                @pl.loop(0, L)
                def _(l):
                    m = (iota == l) & inr
                    plsc.addupdate_scatter(cls_ref, [lx, cy, cls_i],
                                           ones16, mask=m)
                    plsc.addupdate_scatter(obj_ref, [lx, cy], ones16, mask=m)

    handles = []

    @pl.loop(0, 2)
    def _(i):
        @pl.loop(0, 64)
        def _(j):
            for k in range(NUM_CLASSES // L):
                c8_v[i, j, pl.ds(k * L, L)] = zeros16

    for i in range(2):
        for k in range(64 // L):
            o8_v[i, pl.ds(k * L, L)] = zeros16

    _scatter_pass(c8_v, o8_v, 0, 1, x0, 2)
    handles.append(pltpu.async_copy(c8_v, cls8_hbm.at[pl.ds(x0, 2)], sem))
    handles.append(pltpu.async_copy(o8_v, obj8_hbm.at[pl.ds(x0, 2)], sem))

    @pl.loop(0, 32)
    def _(j):
        for k in range(NUM_CLASSES // L):
            c16_v[0, j, pl.ds(k * L, L)] = zeros16

    for k in range(32 // L):
        o16_v[0, pl.ds(k * L, L)] = zeros16

    _scatter_pass(c16_v, o16_v, 2, 3, w, 1)
    handles.append(pltpu.async_copy(c16_v, cls16_hbm.at[pl.ds(w, 1)], sem))
    handles.append(pltpu.async_copy(o16_v, obj16_hbm.at[pl.ds(w, 1)], sem))

    @pl.when(w < 16)
    def _():
        @pl.loop(0, 16)
        def _(j):
            for k in range(NUM_CLASSES // L):
                c32_v[0, j, pl.ds(k * L, L)] = zeros16

        o32_v[0, pl.ds(0, L)] = zeros16

        _scatter_pass(c32_v, o32_v, 4, 5, w, 1)
        pltpu.sync_copy(c32_v, cls32_hbm.at[pl.ds(w, 1)])
        pltpu.sync_copy(o32_v, obj32_hbm.at[pl.ds(w, 1)])

    @pl.when(w == 17)
    def _():
        @pl.loop(0, NVEC)
        def _(v):
            for r in range(4):
                stage_v[r, pl.ds(v * L, L)] = (
                    ci_v[r, pl.ds(v * L, L)].astype(jnp.float32))
        pltpu.sync_copy(stage_v, stage_hbm)

    for h in handles:
        h.wait()


def kernel(image, label):
    (stage, obj8, cls8, obj16, cls16,
     obj32, cls32) = _sc_label_kernel(label)
    pad = jnp.zeros((MAX_BOXES - N, 4), jnp.float32)
    wh = label[:, 3:5]
    box8 = jnp.concatenate([label[:, 1:5], pad], axis=0)
    box16 = jnp.concatenate(
        [jnp.concatenate([stage[0:2, :N].T, wh], axis=1), pad], axis=0)
    box32 = jnp.concatenate(
        [jnp.concatenate([stage[2:4, :N].T, wh], axis=1), pad], axis=0)
    return (image, box8, obj8, cls8, box16, obj16, cls16,
            box32, obj32, cls32)


# speedup vs baseline: 1.0391x; 1.0090x over previous
import dataclasses
import functools

import jax
import jax.numpy as jnp
from jax import lax
from jax.experimental import pallas as pl
from jax.experimental.pallas import tpu as pltpu
from jax.experimental.pallas import tpu_sc as plsc

NUM_CLASSES = 80
MAX_BOXES = 100
N = 60
L = 16
NVEC = 4


def _f32(*shape):
    return jax.ShapeDtypeStruct(shape, jnp.float32)


_MESH = plsc.VectorSubcoreMesh(core_axis_name="c", subcore_axis_name="s")

_CP = pltpu.CompilerParams()
if "needs_layout_passes" in pltpu.CompilerParams.__dataclass_fields__:
    _CP = dataclasses.replace(_CP, needs_layout_passes=False)
if "use_tc_tiling_on_sc" in pltpu.CompilerParams.__dataclass_fields__:
    _CP = dataclasses.replace(_CP, use_tc_tiling_on_sc=True)
_CP = dataclasses.replace(_CP, internal_scratch_in_bytes=1 << 20)


@functools.partial(
    pl.kernel,
    out_type=(
        _f32(4, 64),
        _f32(64, 64), _f32(64, 64, NUM_CLASSES),
        _f32(32, 32), _f32(32, 32, NUM_CLASSES),
        _f32(16, 16), _f32(16, 16, NUM_CLASSES),
    ),
    mesh=_MESH,
    compiler_params=_CP,
    scratch_types=[
        pltpu.VMEM((N, 5), jnp.float32),
        pltpu.VMEM((8, NVEC * L), jnp.int32),
        pltpu.VMEM((2, 64, NUM_CLASSES), jnp.float32),
        pltpu.VMEM((1, 32, NUM_CLASSES), jnp.float32),
        pltpu.VMEM((1, 16, NUM_CLASSES), jnp.float32),
        pltpu.VMEM((2, 64), jnp.float32),
        pltpu.VMEM((1, 32), jnp.float32),
        pltpu.VMEM((1, 16), jnp.float32),
        pltpu.VMEM((4, NVEC * L), jnp.float32),
        pltpu.SemaphoreType.DMA,
    ],
)
def _sc_label_kernel(label_hbm,
                     stage_hbm, obj8_hbm, cls8_hbm,
                     obj16_hbm, cls16_hbm, obj32_hbm, cls32_hbm,
                     lab_v, ci_v, c8_v, c16_v, c32_v, o8_v, o16_v, o32_v,
                     stage_v, sem):
    w = lax.axis_index("s") * 2 + lax.axis_index("c")
    x0 = 2 * w

    iota = lax.iota(jnp.int32, L)
    zeros16 = jnp.zeros((L,), jnp.float32)
    ones16 = jnp.ones((L,), jnp.float32)

    pltpu.sync_copy(label_hbm, lab_v)

    @pl.loop(0, NVEC)
    def _(v):
        rows = jnp.minimum(iota + v * L, N - 1)

        def _col(c):
            return plsc.load_gather(lab_v, [rows, jnp.full((L,), c, jnp.int32)])

        c8x = (_col(1) * 0.125).astype(jnp.int32)
        c8y = (_col(2) * 0.125).astype(jnp.int32)
        c16x = lax.shift_right_logical(c8x, 4)
        c16y = lax.shift_right_logical(c8y, 4)
        ci_v[0, pl.ds(v * L, L)] = c8x
        ci_v[1, pl.ds(v * L, L)] = c8y
        ci_v[2, pl.ds(v * L, L)] = c16x
        ci_v[3, pl.ds(v * L, L)] = c16y
        ci_v[4, pl.ds(v * L, L)] = lax.shift_right_logical(c16x, 5)
        ci_v[5, pl.ds(v * L, L)] = lax.shift_right_logical(c16y, 5)
        ci_v[6, pl.ds(v * L, L)] = _col(0).astype(jnp.int32)

    def _scatter_pass(cls_ref, obj_ref, rx, ry, xlo, nrows):
        @pl.loop(0, NVEC)
        def _(v):
            cx = ci_v[rx, pl.ds(v * L, L)]
            inr = (iota < (N - v * L)) & (cx >= xlo) & (cx < xlo + nrows)

            @pl.when(jnp.any(inr))
            def _():
                cy = ci_v[ry, pl.ds(v * L, L)]
                cls_i = ci_v[6, pl.ds(v * L, L)]
                lx = jnp.minimum(jnp.maximum(cx - xlo, 0), nrows - 1)

                @pl.loop(0, L)
                def _(l):
                    m = (iota == l) & inr
                    plsc.addupdate_scatter(cls_ref, [lx, cy, cls_i],
                                           ones16, mask=m)
                    plsc.addupdate_scatter(obj_ref, [lx, cy], ones16, mask=m)

    handles = []

    @pl.loop(0, 2)
    def _(i):
        @pl.loop(0, 64)
        def _(j):
            for k in range(NUM_CLASSES // L):
                c8_v[i, j, pl.ds(k * L, L)] = zeros16

    for i in range(2):
        for k in range(64 // L):
            o8_v[i, pl.ds(k * L, L)] = zeros16

    _scatter_pass(c8_v, o8_v, 0, 1, x0, 2)
    handles.append(pltpu.async_copy(c8_v, cls8_hbm.at[pl.ds(x0, 2)], sem))
    handles.append(pltpu.async_copy(o8_v, obj8_hbm.at[pl.ds(x0, 2)], sem))

    @pl.loop(0, 32)
    def _(j):
        for k in range(NUM_CLASSES // L):
            c16_v[0, j, pl.ds(k * L, L)] = zeros16

    for k in range(32 // L):
        o16_v[0, pl.ds(k * L, L)] = zeros16

    _scatter_pass(c16_v, o16_v, 2, 3, w, 1)
    handles.append(pltpu.async_copy(c16_v, cls16_hbm.at[pl.ds(w, 1)], sem))
    handles.append(pltpu.async_copy(o16_v, obj16_hbm.at[pl.ds(w, 1)], sem))

    @pl.when(w < 16)
    def _():
        @pl.loop(0, 16)
        def _(j):
            for k in range(NUM_CLASSES // L):
                c32_v[0, j, pl.ds(k * L, L)] = zeros16

        o32_v[0, pl.ds(0, L)] = zeros16

        _scatter_pass(c32_v, o32_v, 4, 5, w, 1)
        pltpu.sync_copy(c32_v, cls32_hbm.at[pl.ds(w, 1)])
        pltpu.sync_copy(o32_v, obj32_hbm.at[pl.ds(w, 1)])

    @pl.when(w == 17)
    def _():
        @pl.loop(0, NVEC)
        def _(v):
            for r in range(4):
                stage_v[r, pl.ds(v * L, L)] = (
                    ci_v[r, pl.ds(v * L, L)].astype(jnp.float32))
        pltpu.sync_copy(stage_v, stage_hbm)

    for h in handles:
        h.wait()


def kernel(image, label):
    (stage, obj8, cls8, obj16, cls16,
     obj32, cls32) = _sc_label_kernel(label)
    pad = jnp.zeros((MAX_BOXES - N, 4), jnp.float32)
    wh = label[:, 3:5]
    box8 = jnp.concatenate([label[:, 1:5], pad], axis=0)
    box16 = jnp.concatenate(
        [jnp.concatenate([stage[0:2, :N].T, wh], axis=1), pad], axis=0)
    box32 = jnp.concatenate(
        [jnp.concatenate([stage[2:4, :N].T, wh], axis=1), pad], axis=0)
    return (image, box8, obj8, cls8, box16, obj16, cls16,
            box32, obj32, cls32)
